# baseline (speedup 1.0000x reference)
import functools

import jax
import jax.numpy as jnp
from jax import lax
from jax.experimental import pallas as pl
from jax.experimental.pallas import tpu as pltpu
from jax.experimental.pallas import tpu_sc as plsc

NUM_EMBEDDINGS = 100000
EMBED_DIM = 128
BATCH = 4096
HIST = 50

NUM_CORES = 2
NUM_SUBCORES = 16
NUM_WORKERS = NUM_CORES * NUM_SUBCORES
BPW = BATCH // NUM_WORKERS
GROUP = 2
NUM_GROUPS = HIST // GROUP
NBUF = 3
ROUNDS = (NUM_GROUPS - 1) // NBUF

_mesh = plsc.VectorSubcoreMesh(
    core_axis_name="c",
    subcore_axis_name="s",
    num_cores=NUM_CORES,
    num_subcores=NUM_SUBCORES,
)


@functools.partial(
    pl.kernel,
    out_type=jax.ShapeDtypeStruct((HIST, BATCH, EMBED_DIM), jnp.float32),
    mesh=_mesh,
    scratch_types=[
        pltpu.VMEM((HIST, BPW), jnp.int32),
        [pltpu.VMEM((GROUP, BPW, EMBED_DIM), jnp.float32)] * NBUF,
        [pltpu.SemaphoreType.DMA] * NBUF,
        [pltpu.SemaphoreType.DMA] * NBUF,
    ],
)
def _gather_kernel(idx_hbm, table_hbm, out_hbm, idx_v, bufs, gsems, wsems):
    wid = lax.axis_index("s") * NUM_CORES + lax.axis_index("c")
    wbase = wid * BPW
    pltpu.sync_copy(idx_hbm.at[:, wid], idx_v)

    def fire_group(g, p):
        for k in range(GROUP):
            pltpu.async_copy(
                table_hbm.at[idx_v.at[g * GROUP + k]], bufs[p].at[k], gsems[p]
            )

    def wait_group(p):
        pltpu.make_async_copy(
            out_hbm.at[pl.ds(0, GROUP), pl.ds(0, BPW)], bufs[p], gsems[p]
        ).wait()

    def write_group(g, p):
        pltpu.async_copy(
            bufs[p], out_hbm.at[pl.ds(g * GROUP, GROUP), pl.ds(wbase, BPW)], wsems[p]
        )

    def wait_write(p):
        pltpu.make_async_copy(
            bufs[p], out_hbm.at[pl.ds(0, GROUP), pl.ds(wbase, BPW)], wsems[p]
        ).wait()

    for p in range(NBUF):
        fire_group(p, p)

    def round_body(r, _):
        for p in range(NBUF):
            g = r * NBUF + p
            wait_group(p)
            write_group(g, p)
        for p in range(NBUF):
            wait_write(p)
            g_next = (r + 1) * NBUF + p

            @pl.when(g_next < NUM_GROUPS)
            def _():
                fire_group(g_next, p)

        return 0

    lax.fori_loop(0, ROUNDS, round_body, 0)

    wait_group(0)
    write_group(NUM_GROUPS - 1, 0)
    wait_write(0)


def kernel(token_ids, table):
    idx = token_ids.astype(jnp.int32).T.reshape(HIST, NUM_WORKERS, BPW)
    out_hbd = _gather_kernel(idx, table)
    return out_hbd.transpose(1, 0, 2)


# speedup vs baseline: 1.0454x; 1.0454x over previous
import functools

import jax
import jax.numpy as jnp
from jax import lax
from jax.experimental import pallas as pl
from jax.experimental.pallas import tpu as pltpu
from jax.experimental.pallas import tpu_sc as plsc

NUM_EMBEDDINGS = 100000
EMBED_DIM = 128
BATCH = 4096
HIST = 50

NUM_CORES = 2
NUM_SUBCORES = 16
NUM_WORKERS = NUM_CORES * NUM_SUBCORES
BPW = BATCH // NUM_WORKERS
NBUF = 5
ROUNDS = HIST // NBUF

_mesh = plsc.VectorSubcoreMesh(
    core_axis_name="c",
    subcore_axis_name="s",
    num_cores=NUM_CORES,
    num_subcores=NUM_SUBCORES,
)


@functools.partial(
    pl.kernel,
    out_type=jax.ShapeDtypeStruct((HIST, BATCH, EMBED_DIM), jnp.float32),
    mesh=_mesh,
    scratch_types=[
        pltpu.VMEM((HIST, BPW), jnp.int32),
        [pltpu.VMEM((BPW, EMBED_DIM), jnp.float32)] * NBUF,
        [pltpu.SemaphoreType.DMA] * NBUF,
        [pltpu.SemaphoreType.DMA] * NBUF,
    ],
)
def _gather_kernel(idx_hbm, table_hbm, out_hbm, idx_v, bufs, gsems, wsems):
    wid = lax.axis_index("s") * NUM_CORES + lax.axis_index("c")
    wbase = wid * BPW
    pltpu.sync_copy(idx_hbm.at[:, wid], idx_v)

    for h in range(NBUF):
        pltpu.async_copy(table_hbm.at[idx_v.at[h]], bufs[h], gsems[h])

    for h in range(HIST):
        b = h % NBUF
        pltpu.make_async_copy(table_hbm.at[idx_v.at[0]], bufs[b], gsems[b]).wait()
        pltpu.async_copy(bufs[b], out_hbm.at[h, pl.ds(wbase, BPW)], wsems[b])
        if h + NBUF < HIST:
            pltpu.make_async_copy(
                bufs[b], out_hbm.at[0, pl.ds(wbase, BPW)], wsems[b]
            ).wait()
            pltpu.async_copy(
                table_hbm.at[idx_v.at[h + NBUF]], bufs[b], gsems[b]
            )

    for h in range(HIST - NBUF, HIST):
        b = h % NBUF
        pltpu.make_async_copy(
            bufs[b], out_hbm.at[0, pl.ds(wbase, BPW)], wsems[b]
        ).wait()


def kernel(token_ids, table):
    idx = token_ids.astype(jnp.int32).T.reshape(HIST, NUM_WORKERS, BPW)
    out_hbd = _gather_kernel(idx, table)
    return out_hbd.transpose(1, 0, 2)


# speedup vs baseline: 1.0576x; 1.0117x over previous
import functools

import jax
import jax.numpy as jnp
from jax import lax
from jax.experimental import pallas as pl
from jax.experimental.pallas import tpu as pltpu
from jax.experimental.pallas import tpu_sc as plsc

NUM_EMBEDDINGS = 100000
EMBED_DIM = 128
BATCH = 4096
HIST = 50

NUM_CORES = 2
NUM_SUBCORES = 16
NUM_WORKERS = NUM_CORES * NUM_SUBCORES
BPW = BATCH // NUM_WORKERS
NBUF = 7

_mesh = plsc.VectorSubcoreMesh(
    core_axis_name="c",
    subcore_axis_name="s",
    num_cores=NUM_CORES,
    num_subcores=NUM_SUBCORES,
)


@functools.partial(
    pl.kernel,
    out_type=jax.ShapeDtypeStruct((HIST, BATCH, EMBED_DIM), jnp.float32),
    mesh=_mesh,
    scratch_types=[
        pltpu.VMEM((HIST, BPW), jnp.int32),
        [pltpu.VMEM((BPW, EMBED_DIM), jnp.float32)] * NBUF,
        [pltpu.SemaphoreType.DMA] * NBUF,
        [pltpu.SemaphoreType.DMA] * NBUF,
    ],
)
def _gather_kernel(idx_hbm, table_hbm, out_hbm, idx_v, bufs, gsems, wsems):
    wid = lax.axis_index("s") * NUM_CORES + lax.axis_index("c")
    wbase = wid * BPW
    pltpu.sync_copy(idx_hbm.at[:, wid], idx_v)

    for h in range(NBUF):
        pltpu.async_copy(table_hbm.at[idx_v.at[h]], bufs[h], gsems[h])

    for h in range(HIST):
        b = h % NBUF
        pltpu.make_async_copy(table_hbm.at[idx_v.at[0]], bufs[b], gsems[b]).wait()
        pltpu.async_copy(bufs[b], out_hbm.at[h, pl.ds(wbase, BPW)], wsems[b])
        if h + NBUF < HIST:
            pltpu.make_async_copy(
                bufs[b], out_hbm.at[0, pl.ds(wbase, BPW)], wsems[b]
            ).wait()
            pltpu.async_copy(
                table_hbm.at[idx_v.at[h + NBUF]], bufs[b], gsems[b]
            )

    for h in range(HIST - NBUF, HIST):
        b = h % NBUF
        pltpu.make_async_copy(
            bufs[b], out_hbm.at[0, pl.ds(wbase, BPW)], wsems[b]
        ).wait()


def kernel(token_ids, table):
    idx = token_ids.astype(jnp.int32).T.reshape(HIST, NUM_WORKERS, BPW)
    out_hbd = _gather_kernel(idx, table)
    return out_hbd.transpose(1, 0, 2)
